# baseline (speedup 1.0000x reference)
import functools
import math

import jax
import jax.numpy as jnp
from jax import lax
from jax.experimental import pallas as pl
from jax.experimental.pallas import tpu as pltpu
from jax.experimental.pallas import tpu_sc as plsc

VOCAB = 100000
D = 64
BATCH = 4096
SEQ = 20
B = BATCH * SEQ

NUM_CORES = 2
NUM_SUBCORES = 16
LANES = 16
TPERX = 128
TOK_PER_TILE = B // NUM_SUBCORES
SEQ_PER_TILE = BATCH // NUM_SUBCORES
D_PER_SC = D // NUM_CORES
D_PASSES = D_PER_SC // NUM_SUBCORES
GRP = 8


def _positional_encodings():
    position = jnp.arange(SEQ, dtype=jnp.float32)[:, None]
    div_term = jnp.exp(
        jnp.arange(0, D, 2, dtype=jnp.float32) * -(math.log(10000.0) / D)
    )
    wpe = jnp.zeros((SEQ, D), dtype=jnp.float32)
    wpe = wpe.at[:, 0::2].set(jnp.sin(position * div_term))
    wpe = wpe.at[:, 1::2].set(jnp.cos(position * div_term))
    return wpe


def _sc_kernel(ids_hbm, remap_hbm, table_hbm, wpe_hbm, out_hbm,
               row_v, ibuf, rbuf, idsb, obufb, wpe_v,
               dids_sh, rsem, ids_sem, osem0, osem1, gsem, tsem0):
    cid = lax.axis_index("c")
    sid = lax.axis_index("s")

    tok0 = sid * TOK_PER_TILE
    half = TOK_PER_TILE // 2

    with jax.named_scope("ph1_remap0"):
        icp0 = pltpu.async_copy(ids_hbm.at[pl.ds(tok0, half)], ibuf, tsem0)
        icp1 = pltpu.async_copy(ids_hbm.at[pl.ds(tok0 + half, half)],
                                idsb.at[pl.ds(0, half)], ids_sem)
        icp0.wait()
        cp0 = pltpu.async_copy(remap_hbm.at[ibuf], rbuf.at[pl.ds(0, half)], gsem)
        cp0.wait()
        d0 = cid * D_PER_SC + sid
        pltpu.async_copy(table_hbm.at[d0], row_v, rsem)
    with jax.named_scope("ph1_rest"):
        icp1.wait()
        cp1 = pltpu.async_copy(remap_hbm.at[idsb.at[pl.ds(0, half)]],
                               rbuf.at[pl.ds(half, half)], gsem)
        tcp = pltpu.async_copy(rbuf.at[pl.ds(0, half)],
                               dids_sh.at[pl.ds(tok0, half)], tsem0)
        cp1.wait()
        pltpu.sync_copy(rbuf.at[pl.ds(half, half)],
                        dids_sh.at[pl.ds(tok0 + half, half)])
        tcp.wait()
    with jax.named_scope("ph1_barrier"):
        plsc.subcore_barrier()

    def one_pass(p, _):
        d = cid * D_PER_SC + sid + NUM_SUBCORES * p
        with jax.named_scope("row_wait"):
            pltpu.make_async_copy(table_hbm.at[d], row_v, rsem).wait()
        pltpu.sync_copy(wpe_hbm.at[pl.ds(d * SEQ * LANES, SEQ * LANES)], wpe_v)
        pltpu.async_copy(dids_sh.at[pl.ds(0, BATCH)],
                         idsb.at[pl.ds(0, BATCH)], ids_sem)

        def seq_step(s, _):
            par = s % 2
            ib = par * BATCH
            pltpu.make_async_copy(dids_sh.at[pl.ds(s * BATCH, BATCH)],
                                  idsb.at[pl.ds(ib, BATCH)], ids_sem).wait()

            @pl.when(s + 1 < SEQ)
            def _():
                pltpu.async_copy(dids_sh.at[pl.ds((s + 1) * BATCH, BATCH)],
                                 idsb.at[pl.ds(BATCH - ib, BATCH)], ids_sem)

            @pl.when((s >= 2) & (par == 0))
            def _():
                pltpu.make_async_copy(obufb.at[pl.ds(0, BATCH)],
                                      out_hbm.at[s - 2, d], osem0).wait()

            @pl.when((s >= 2) & (par == 1))
            def _():
                pltpu.make_async_copy(obufb.at[pl.ds(BATCH, BATCH)],
                                      out_hbm.at[s - 2, d], osem1).wait()

            w = wpe_v[pl.ds(s * LANES, LANES)]

            def loads(g):
                base = ib + g * (GRP * LANES)
                return [plsc.load_gather(
                            row_v, [idsb[pl.ds(base + j * LANES, LANES)]]) + w
                        for j in range(GRP)]

            ngrp = BATCH // LANES // GRP

            def grp_step(g, vals):
                nxt = loads(g)
                base = ib + (g - 1) * (GRP * LANES)
                for j in range(GRP):
                    obufb[pl.ds(base + j * LANES, LANES)] = vals[j]
                return tuple(nxt)

            vals = lax.fori_loop(1, ngrp, grp_step, tuple(loads(0)))
            for j in range(GRP):
                obufb[pl.ds(ib + (ngrp - 1) * (GRP * LANES) + j * LANES, LANES)] = vals[j]

            @pl.when(par == 0)
            def _():
                pltpu.async_copy(obufb.at[pl.ds(0, BATCH)], out_hbm.at[s, d], osem0)

            @pl.when(par == 1)
            def _():
                pltpu.async_copy(obufb.at[pl.ds(BATCH, BATCH)], out_hbm.at[s, d], osem1)
            return 0

        with jax.named_scope("sloop"):
            lax.fori_loop(0, SEQ, seq_step, 0)

        @pl.when(p + 1 < D_PASSES)
        def _():
            pltpu.async_copy(table_hbm.at[d + NUM_SUBCORES], row_v, rsem)

        pltpu.make_async_copy(obufb.at[pl.ds(0, BATCH)],
                              out_hbm.at[SEQ - 2, d], osem0).wait()
        pltpu.make_async_copy(obufb.at[pl.ds(BATCH, BATCH)],
                              out_hbm.at[SEQ - 1, d], osem1).wait()
        return 0

    lax.fori_loop(0, D_PASSES, one_pass, 0)


@jax.jit
def _run(ids_flat, remap, table_t, wpe_dmaj):
    mesh = plsc.VectorSubcoreMesh(core_axis_name="c", subcore_axis_name="s")
    call = functools.partial(
        pl.kernel,
        mesh=mesh,
        out_type=jax.ShapeDtypeStruct((SEQ, D, BATCH), jnp.float32),
        scratch_types=[
            pltpu.VMEM((VOCAB,), jnp.float32),
            pltpu.VMEM((TOK_PER_TILE // 2,), jnp.int32),
            pltpu.VMEM((TOK_PER_TILE,), jnp.int32),
            pltpu.VMEM((2 * BATCH,), jnp.int32),
            pltpu.VMEM((2 * BATCH,), jnp.float32),
            pltpu.VMEM((SEQ * LANES,), jnp.float32),
            pltpu.VMEM_SHARED((B,), jnp.int32),
            pltpu.SemaphoreType.DMA,
            pltpu.SemaphoreType.DMA,
            pltpu.SemaphoreType.DMA,
            pltpu.SemaphoreType.DMA,
            pltpu.SemaphoreType.DMA,
            pltpu.SemaphoreType.DMA,
        ],
        compiler_params=pltpu.CompilerParams(use_tc_tiling_on_sc=True,
                                             needs_layout_passes=False),
    )(_sc_kernel)
    return call(ids_flat, remap, table_t, wpe_dmaj)


def kernel(input_ids, embedding_dict, input_ids2dict_ids):
    ids_flat = input_ids.T.reshape(-1).astype(jnp.int32)
    remap = input_ids2dict_ids.astype(jnp.int32)
    table_t = embedding_dict.T
    wpe = _positional_encodings()
    wpe_dmaj = jnp.broadcast_to(wpe.T.reshape(D, SEQ, 1), (D, SEQ, LANES)).reshape(-1)
    out_t = _run(ids_flat, remap, table_t, wpe_dmaj)
    return out_t.transpose(2, 0, 1)


# speedup vs baseline: 1.0914x; 1.0914x over previous
import functools
import math

import jax
import jax.numpy as jnp
from jax import lax
from jax.experimental import pallas as pl
from jax.experimental.pallas import tpu as pltpu
from jax.experimental.pallas import tpu_sc as plsc

VOCAB = 100000
D = 64
BATCH = 4096
SEQ = 20
B = BATCH * SEQ

NUM_CORES = 2
NUM_SUBCORES = 16
LANES = 16
TPERX = 128
TOK_PER_TILE = B // NUM_SUBCORES
SEQ_PER_TILE = BATCH // NUM_SUBCORES
D_PER_SC = D // NUM_CORES
D_PASSES = D_PER_SC // NUM_SUBCORES
GRP = 8


def _positional_encodings():
    position = jnp.arange(SEQ, dtype=jnp.float32)[:, None]
    div_term = jnp.exp(
        jnp.arange(0, D, 2, dtype=jnp.float32) * -(math.log(10000.0) / D)
    )
    wpe = jnp.zeros((SEQ, D), dtype=jnp.float32)
    wpe = wpe.at[:, 0::2].set(jnp.sin(position * div_term))
    wpe = wpe.at[:, 1::2].set(jnp.cos(position * div_term))
    return wpe


def _sc_kernel(ids_hbm, remap_hbm, table_hbm, wpe_hbm, out_hbm,
               row_v, ibuf, rbuf, ids0, ids1, obuf0, obuf1, wpe_v,
               dids_sh, rsem, ids_sem, osem0, osem1, gsem, tsem0):
    cid = lax.axis_index("c")
    sid = lax.axis_index("s")

    tok0 = sid * TOK_PER_TILE
    half = TOK_PER_TILE // 2

    with jax.named_scope("ph1_remap0"):
        icp0 = pltpu.async_copy(ids_hbm.at[pl.ds(tok0, half)], ibuf, tsem0)
        icp1 = pltpu.async_copy(ids_hbm.at[pl.ds(tok0 + half, half)],
                                ids0.at[pl.ds(0, half)], ids_sem)
        icp0.wait()
        cp0 = pltpu.async_copy(remap_hbm.at[ibuf], rbuf.at[pl.ds(0, half)], gsem)
        cp0.wait()
        d0 = cid * D_PER_SC + sid
        pltpu.async_copy(table_hbm.at[d0], row_v, rsem)
    with jax.named_scope("ph1_rest"):
        icp1.wait()
        cp1 = pltpu.async_copy(remap_hbm.at[ids0.at[pl.ds(0, half)]],
                               rbuf.at[pl.ds(half, half)], gsem)
        tcp = pltpu.async_copy(rbuf.at[pl.ds(0, half)],
                               dids_sh.at[pl.ds(tok0, half)], tsem0)
        cp1.wait()
        pltpu.sync_copy(rbuf.at[pl.ds(half, half)],
                        dids_sh.at[pl.ds(tok0 + half, half)])
        tcp.wait()
    with jax.named_scope("ph1_barrier"):
        plsc.subcore_barrier()

    def one_pass(p, _):
        d = cid * D_PER_SC + sid + NUM_SUBCORES * p
        with jax.named_scope("row_wait"):
            pltpu.make_async_copy(table_hbm.at[d], row_v, rsem).wait()
        pltpu.sync_copy(wpe_hbm.at[pl.ds(d * SEQ * LANES, SEQ * LANES)], wpe_v)
        pltpu.async_copy(dids_sh.at[pl.ds(0, BATCH)], ids0, ids_sem)

        def body(s, ids_cur, ids_nxt, obuf, osem):
            pltpu.make_async_copy(dids_sh.at[pl.ds(s * BATCH, BATCH)],
                                  ids_cur, ids_sem).wait()

            @pl.when(s + 1 < SEQ)
            def _():
                pltpu.async_copy(dids_sh.at[pl.ds((s + 1) * BATCH, BATCH)],
                                 ids_nxt, ids_sem)

            @pl.when(s >= 2)
            def _():
                pltpu.make_async_copy(obuf, out_hbm.at[s - 2, d], osem).wait()

            w = wpe_v[pl.ds(s * LANES, LANES)]

            def loads(g):
                base = g * (GRP * LANES)
                return [plsc.load_gather(
                            row_v, [ids_cur[pl.ds(base + j * LANES, LANES)]]) + w
                        for j in range(GRP)]

            ngrp = BATCH // LANES // GRP

            def grp_step(g, vals):
                nxt = loads(g)
                base = (g - 1) * (GRP * LANES)
                for j in range(GRP):
                    obuf[pl.ds(base + j * LANES, LANES)] = vals[j]
                return tuple(nxt)

            vals = lax.fori_loop(1, ngrp, grp_step, tuple(loads(0)))
            for j in range(GRP):
                obuf[pl.ds((ngrp - 1) * (GRP * LANES) + j * LANES, LANES)] = vals[j]
            pltpu.async_copy(obuf, out_hbm.at[s, d], osem)

        def seq_step(s, _):
            @pl.when(s % 2 == 0)
            def _():
                body(s, ids0, ids1, obuf0, osem0)

            @pl.when(s % 2 == 1)
            def _():
                body(s, ids1, ids0, obuf1, osem1)
            return 0

        with jax.named_scope("sloop"):
            lax.fori_loop(0, SEQ, seq_step, 0)

        @pl.when(p + 1 < D_PASSES)
        def _():
            pltpu.async_copy(table_hbm.at[d + NUM_SUBCORES], row_v, rsem)

        pltpu.make_async_copy(obuf0, out_hbm.at[SEQ - 2, d], osem0).wait()
        pltpu.make_async_copy(obuf1, out_hbm.at[SEQ - 1, d], osem1).wait()
        return 0

    lax.fori_loop(0, D_PASSES, one_pass, 0)


@jax.jit
def _run(ids_flat, remap, table_t, wpe_dmaj):
    mesh = plsc.VectorSubcoreMesh(core_axis_name="c", subcore_axis_name="s")
    call = functools.partial(
        pl.kernel,
        mesh=mesh,
        out_type=jax.ShapeDtypeStruct((SEQ, D, BATCH), jnp.float32),
        scratch_types=[
            pltpu.VMEM((VOCAB,), jnp.float32),
            pltpu.VMEM((TOK_PER_TILE // 2,), jnp.int32),
            pltpu.VMEM((TOK_PER_TILE,), jnp.int32),
            pltpu.VMEM((BATCH,), jnp.int32),
            pltpu.VMEM((BATCH,), jnp.int32),
            pltpu.VMEM((BATCH,), jnp.float32),
            pltpu.VMEM((BATCH,), jnp.float32),
            pltpu.VMEM((SEQ * LANES,), jnp.float32),
            pltpu.VMEM_SHARED((B,), jnp.int32),
            pltpu.SemaphoreType.DMA,
            pltpu.SemaphoreType.DMA,
            pltpu.SemaphoreType.DMA,
            pltpu.SemaphoreType.DMA,
            pltpu.SemaphoreType.DMA,
            pltpu.SemaphoreType.DMA,
        ],
        compiler_params=pltpu.CompilerParams(use_tc_tiling_on_sc=True,
                                             needs_layout_passes=False),
    )(_sc_kernel)
    return call(ids_flat, remap, table_t, wpe_dmaj)


def kernel(input_ids, embedding_dict, input_ids2dict_ids):
    ids_flat = input_ids.T.reshape(-1).astype(jnp.int32)
    remap = input_ids2dict_ids.astype(jnp.int32)
    table_t = embedding_dict.T
    wpe = _positional_encodings()
    wpe_dmaj = jnp.broadcast_to(wpe.T.reshape(D, SEQ, 1), (D, SEQ, LANES)).reshape(-1)
    out_t = _run(ids_flat, remap, table_t, wpe_dmaj)
    return out_t.transpose(2, 0, 1)
